# Initial kernel scaffold; baseline (speedup 1.0000x reference)
import jax
import jax.numpy as jnp
from jax.experimental import pallas as pl


def kernel(x, error, w, edge_index):
    raise NotImplementedError("write your pallas kernel here")


# speedup vs baseline: 95.5976x; 95.5976x over previous
import functools

import jax
import jax.numpy as jnp
from jax import lax
from jax.experimental import pallas as pl
from jax.experimental.pallas import tpu as pltpu
from jax.experimental.pallas import tpu_sc as plsc

_NC = 2
_NS = 16
_NW = _NC * _NS
_L = 16
_C = 128


def _sc_edge_kernel(N, NR, EW):
    rows = NR // _NS
    wclamp = N * N - 1

    K = EW // _C
    assert K % 2 == 0 and K >= 4

    def body(xT, eT, wflat, src, dst, mu_out, ag_out,
             t_sh, e_sh, mu_sh, ag_sh,
             tb, src_v, dst_v, wi_v, we_v,
             t_rows, e_rows, mu_c, ag_c, semG0):
        cid = lax.axis_index("c")
        sid = lax.axis_index("s")
        wid = sid * _NC + cid
        r0 = sid * rows

        pltpu.sync_copy(xT.at[pl.ds(r0, rows)], tb)

        def p0(i, carry):
            v = tb[i, :]
            tb[i, :] = 1.0 - 2.0 / (jnp.exp(2.0 * v) + 1.0)
            return carry
        lax.fori_loop(0, rows, p0, 0)
        pltpu.sync_copy(tb, t_sh.at[pl.ds(r0, rows)])
        pltpu.sync_copy(eT.at[pl.ds(r0, rows)], tb)
        pltpu.sync_copy(tb, e_sh.at[pl.ds(r0, rows)])

        def z0(i, carry):
            tb[i, :] = jnp.zeros((_L,), jnp.float32)
            return carry
        lax.fori_loop(0, rows, z0, 0)
        pltpu.sync_copy(tb, mu_sh.at[pl.ds(r0, rows)])
        pltpu.sync_copy(tb, ag_sh.at[pl.ds(r0, rows)])
        plsc.subcore_barrier()

        eb = wid * (EW + 2 * _C)

        def chunk(k, carry):
            off = eb + k * _C
            pltpu.sync_copy(src.at[pl.ds(off, _C)], src_v)
            pltpu.sync_copy(dst.at[pl.ds(off, _C)], dst_v)

            def widx(j, carry2):
                sv = src_v[pl.ds(j * _L, _L)]
                dv = dst_v[pl.ds(j * _L, _L)]
                wi_v[pl.ds(j * _L, _L)] = jnp.minimum(sv * N + dv, wclamp)
                return carry2
            lax.fori_loop(0, _C // _L, widx, 0)

            pltpu.async_copy(wflat.at[wi_v], we_v, semG0).wait()
            pltpu.async_copy(t_sh.at[dst_v], t_rows, semG0).wait()
            pltpu.async_copy(e_sh.at[src_v], e_rows, semG0).wait()

            def edge16(j, carry2):
                base = j * _L
                wv16 = we_v[pl.ds(base, _L)]
                for l in range(_L):
                    wv = wv16[l]
                    t = t_rows[base + l, :]
                    e = e_rows[base + l, :]
                    mu_c[base + l, :] = t * wv
                    ag_c[base + l, :] = (1.0 - t * t) * (e * wv)
                return carry2
            lax.fori_loop(0, _C // _L, edge16, 0)

            pltpu.sync_copy(mu_c, mu_sh.at[src_v], add=True)
            pltpu.sync_copy(ag_c, ag_sh.at[dst_v], add=True)
            return carry
        lax.fori_loop(0, K, chunk, 0)
        plsc.subcore_barrier()

        pltpu.sync_copy(mu_sh.at[pl.ds(r0, rows)], tb)
        pltpu.sync_copy(tb, mu_out.at[cid, pl.ds(r0, rows)])
        pltpu.sync_copy(ag_sh.at[pl.ds(r0, rows)], tb)
        pltpu.sync_copy(tb, ag_out.at[cid, pl.ds(r0, rows)])

    return pl.kernel(
        body,
        out_type=[jax.ShapeDtypeStruct((_NC, NR, _L), jnp.float32)] * 2,
        mesh=plsc.VectorSubcoreMesh(core_axis_name="c", subcore_axis_name="s",
                                    num_cores=_NC, num_subcores=_NS),
        compiler_params=pltpu.CompilerParams(use_tc_tiling_on_sc=False),
        scratch_types=[
            pltpu.VMEM_SHARED((NR, _L), jnp.float32),
            pltpu.VMEM_SHARED((NR, _L), jnp.float32),
            pltpu.VMEM_SHARED((NR, _L), jnp.float32),
            pltpu.VMEM_SHARED((NR, _L), jnp.float32),
            pltpu.VMEM((NR // _NS, _L), jnp.float32),
            pltpu.VMEM((_C,), jnp.int32),
            pltpu.VMEM((_C,), jnp.int32),
            pltpu.VMEM((_C,), jnp.int32),
            pltpu.VMEM((_C,), jnp.float32),
            pltpu.VMEM((_C, _L), jnp.float32),
            pltpu.VMEM((_C, _L), jnp.float32),
            pltpu.VMEM((_C, _L), jnp.float32),
            pltpu.VMEM((_C, _L), jnp.float32),
            pltpu.SemaphoreType.DMA,
        ],
    )


def _combine_body(mu_p, ag_p, e2, mu_o, gr_o):
    mu_o[...] = mu_p[0] + mu_p[1]
    gr_o[...] = e2[...] - (ag_p[0] + ag_p[1])


def kernel(x, error, w, edge_index):
    N = w.shape[0]
    BN = x.shape[0]
    B = BN // N
    E = edge_index.shape[1]
    assert B == _L
    NR = -(-(N + 1) // (_NS * 8)) * (_NS * 8)
    EW = -(-E // _NW)
    EW = -(-EW // _C) * _C
    E_pad = EW * _NW
    pad = E_pad - E

    xT = jnp.pad(x.reshape(B, N).T, ((0, NR - N), (0, 0)))
    eT = jnp.pad(error.reshape(B, N).T, ((0, NR - N), (0, 0)))
    wflat = w.reshape(-1)
    padv = jnp.full((pad,), N, jnp.int32)
    tail = jnp.full((_NW, 2 * _C), N, jnp.int32)
    src = jnp.concatenate(
        [jnp.concatenate([edge_index[0], padv]).reshape(_NW, EW), tail],
        axis=1).reshape(-1)
    dst = jnp.concatenate(
        [jnp.concatenate([edge_index[1], padv]).reshape(_NW, EW), tail],
        axis=1).reshape(-1)

    mu_p, ag_p = _sc_edge_kernel(N, NR, EW)(xT, eT, wflat, src, dst)

    F = NR * _L
    assert F % 128 == 0
    R = F // 128
    muT, gradT = pl.pallas_call(
        _combine_body,
        out_shape=[jax.ShapeDtypeStruct((R, 128), jnp.float32)] * 2,
    )(mu_p.reshape(_NC, R, 128), ag_p.reshape(_NC, R, 128),
      eT.reshape(R, 128))

    mu = muT.reshape(NR, _L)[:N].T.reshape(BN, 1)
    grad = gradT.reshape(NR, _L)[:N].T.reshape(BN, 1)
    return mu, grad


# speedup vs baseline: 102.8836x; 1.0762x over previous
import functools

import jax
import jax.numpy as jnp
from jax import lax
from jax.experimental import pallas as pl
from jax.experimental.pallas import tpu as pltpu
from jax.experimental.pallas import tpu_sc as plsc

_NC = 2
_NS = 16
_NW = _NC * _NS
_L = 16
_C = 128
_CB = 512


def _sc_edge_kernel(N, NR, EW):
    rows = NR // _NS
    wclamp = N * N - 1

    K = EW // _CB
    assert EW % _CB == 0 and K >= 2

    def body(xT, eT, wflat, src, dst, mu_out, ag_out,
             t_sh, e_sh, mu_sh, ag_sh,
             tb, src_v, dst_v, wi_v, we_v, src_s, dst_s,
             t_rows, e_rows, mu_c, ag_c, semG0):
        cid = lax.axis_index("c")
        sid = lax.axis_index("s")
        wid = sid * _NC + cid
        r0 = sid * rows

        pltpu.sync_copy(xT.at[pl.ds(r0, rows)], tb)

        def p0(i, carry):
            v = tb[i, :]
            tb[i, :] = 1.0 - 2.0 / (jnp.exp(2.0 * v) + 1.0)
            return carry
        lax.fori_loop(0, rows, p0, 0)
        pltpu.sync_copy(tb, t_sh.at[pl.ds(r0, rows)])
        pltpu.sync_copy(eT.at[pl.ds(r0, rows)], tb)
        pltpu.sync_copy(tb, e_sh.at[pl.ds(r0, rows)])

        def z0(i, carry):
            tb[i, :] = jnp.zeros((_L,), jnp.float32)
            return carry
        lax.fori_loop(0, rows, z0, 0)
        pltpu.sync_copy(tb, mu_sh.at[pl.ds(r0, rows)])
        pltpu.sync_copy(tb, ag_sh.at[pl.ds(r0, rows)])
        plsc.subcore_barrier()

        eb = wid * (EW + 2 * _C)

        def chunk(k, carry):
            off = eb + k * _CB
            pltpu.sync_copy(src.at[pl.ds(off, _CB)], src_v)
            pltpu.sync_copy(dst.at[pl.ds(off, _CB)], dst_v)

            for jr in range(_CB // _C):
                def widx(j2, carry2, jr=jr):
                    sv = src_v[pl.ds(jr * _C + j2 * _L, _L)]
                    dv = dst_v[pl.ds(jr * _C + j2 * _L, _L)]
                    wi_v[pl.ds(jr * _C + j2 * _L, _L)] = (
                        jnp.minimum(sv * N + dv, wclamp))
                    src_s[jr, pl.ds(j2 * _L, _L)] = sv
                    dst_s[jr, pl.ds(j2 * _L, _L)] = dv
                    return carry2
                lax.fori_loop(0, _C // _L, widx, 0)

            pltpu.async_copy(wflat.at[wi_v], we_v, semG0).wait()
            pltpu.async_copy(t_sh.at[dst_v], t_rows, semG0).wait()
            pltpu.async_copy(e_sh.at[src_v], e_rows, semG0).wait()

            for jr in range(_CB // _C):
                def edge16(j2, carry2, jr=jr):
                    base = jr * _C + j2 * _L
                    wv16 = we_v[pl.ds(base, _L)]
                    for l in range(_L):
                        wv = wv16[l]
                        t = t_rows[base + l, :]
                        e = e_rows[base + l, :]
                        mu_c[jr, j2 * _L + l, :] = t * wv
                        ag_c[jr, j2 * _L + l, :] = (1.0 - t * t) * (e * wv)
                    return carry2
                lax.fori_loop(0, _C // _L, edge16, 0)
                pltpu.sync_copy(mu_c.at[jr], mu_sh.at[src_s.at[jr]],
                                add=True)
                pltpu.sync_copy(ag_c.at[jr], ag_sh.at[dst_s.at[jr]],
                                add=True)
            return carry
        lax.fori_loop(0, K, chunk, 0)
        plsc.subcore_barrier()

        pltpu.sync_copy(mu_sh.at[pl.ds(r0, rows)], tb)
        pltpu.sync_copy(tb, mu_out.at[cid, pl.ds(r0, rows)])
        pltpu.sync_copy(ag_sh.at[pl.ds(r0, rows)], tb)
        pltpu.sync_copy(tb, ag_out.at[cid, pl.ds(r0, rows)])

    return pl.kernel(
        body,
        out_type=[jax.ShapeDtypeStruct((_NC, NR, _L), jnp.float32)] * 2,
        mesh=plsc.VectorSubcoreMesh(core_axis_name="c", subcore_axis_name="s",
                                    num_cores=_NC, num_subcores=_NS),
        compiler_params=pltpu.CompilerParams(use_tc_tiling_on_sc=False),
        scratch_types=[
            pltpu.VMEM_SHARED((NR, _L), jnp.float32),
            pltpu.VMEM_SHARED((NR, _L), jnp.float32),
            pltpu.VMEM_SHARED((NR, _L), jnp.float32),
            pltpu.VMEM_SHARED((NR, _L), jnp.float32),
            pltpu.VMEM((NR // _NS, _L), jnp.float32),
            pltpu.VMEM((_CB,), jnp.int32),
            pltpu.VMEM((_CB,), jnp.int32),
            pltpu.VMEM((_CB,), jnp.int32),
            pltpu.VMEM((_CB,), jnp.float32),
            pltpu.VMEM((_CB // _C, _C), jnp.int32),
            pltpu.VMEM((_CB // _C, _C), jnp.int32),
            pltpu.VMEM((_CB, _L), jnp.float32),
            pltpu.VMEM((_CB, _L), jnp.float32),
            pltpu.VMEM((_CB // _C, _C, _L), jnp.float32),
            pltpu.VMEM((_CB // _C, _C, _L), jnp.float32),
            pltpu.SemaphoreType.DMA,
        ],
    )


def _combine_body(mu_p, ag_p, e2, mu_o, gr_o):
    mu_o[...] = mu_p[0] + mu_p[1]
    gr_o[...] = e2[...] - (ag_p[0] + ag_p[1])


def kernel(x, error, w, edge_index):
    N = w.shape[0]
    BN = x.shape[0]
    B = BN // N
    E = edge_index.shape[1]
    assert B == _L
    NR = -(-(N + 1) // (_NS * 8)) * (_NS * 8)
    EW = -(-E // _NW)
    EW = -(-EW // _CB) * _CB
    E_pad = EW * _NW
    pad = E_pad - E

    xT = jnp.pad(x.reshape(B, N).T, ((0, NR - N), (0, 0)))
    eT = jnp.pad(error.reshape(B, N).T, ((0, NR - N), (0, 0)))
    wflat = w.reshape(-1)
    padv = jnp.full((pad,), N, jnp.int32)
    tail = jnp.full((_NW, 2 * _C), N, jnp.int32)
    src = jnp.concatenate(
        [jnp.concatenate([edge_index[0], padv]).reshape(_NW, EW), tail],
        axis=1).reshape(-1)
    dst = jnp.concatenate(
        [jnp.concatenate([edge_index[1], padv]).reshape(_NW, EW), tail],
        axis=1).reshape(-1)

    mu_p, ag_p = _sc_edge_kernel(N, NR, EW)(xT, eT, wflat, src, dst)

    F = NR * _L
    assert F % 128 == 0
    R = F // 128
    muT, gradT = pl.pallas_call(
        _combine_body,
        out_shape=[jax.ShapeDtypeStruct((R, 128), jnp.float32)] * 2,
    )(mu_p.reshape(_NC, R, 128), ag_p.reshape(_NC, R, 128),
      eT.reshape(R, 128))

    mu = muT.reshape(NR, _L)[:N].T.reshape(BN, 1)
    grad = gradT.reshape(NR, _L)[:N].T.reshape(BN, 1)
    return mu, grad


# speedup vs baseline: 104.8536x; 1.0191x over previous
import functools

import jax
import jax.numpy as jnp
from jax import lax
from jax.experimental import pallas as pl
from jax.experimental.pallas import tpu as pltpu
from jax.experimental.pallas import tpu_sc as plsc

_NC = 2
_NS = 16
_NW = _NC * _NS
_L = 16
_C = 512
_CB = 1024


def _sc_edge_kernel(N, NR, EW):
    rows = NR // _NS
    wclamp = N * N - 1

    K = EW // _CB
    assert EW % _CB == 0 and K >= 2

    def body(xT, eT, wflat, src, dst, mu_out, ag_out,
             t_sh, e_sh, mu_sh, ag_sh,
             tb, src_v, dst_v, wi_v, we_v, src_s, dst_s,
             t_rows, e_rows, mu_c, ag_c, semG0):
        cid = lax.axis_index("c")
        sid = lax.axis_index("s")
        wid = sid * _NC + cid
        r0 = sid * rows

        pltpu.sync_copy(xT.at[pl.ds(r0, rows)], tb)

        def p0(i, carry):
            v = tb[i, :]
            tb[i, :] = 1.0 - 2.0 / (jnp.exp(2.0 * v) + 1.0)
            return carry
        lax.fori_loop(0, rows, p0, 0)
        pltpu.sync_copy(tb, t_sh.at[pl.ds(r0, rows)])
        pltpu.sync_copy(eT.at[pl.ds(r0, rows)], tb)
        pltpu.sync_copy(tb, e_sh.at[pl.ds(r0, rows)])

        def z0(i, carry):
            tb[i, :] = jnp.zeros((_L,), jnp.float32)
            return carry
        lax.fori_loop(0, rows, z0, 0)
        pltpu.sync_copy(tb, mu_sh.at[pl.ds(r0, rows)])
        pltpu.sync_copy(tb, ag_sh.at[pl.ds(r0, rows)])
        plsc.subcore_barrier()

        eb = wid * (EW + 2 * _C)

        def chunk(k, carry):
            off = eb + k * _CB
            pltpu.sync_copy(src.at[pl.ds(off, _CB)], src_v)
            pltpu.sync_copy(dst.at[pl.ds(off, _CB)], dst_v)

            for jr in range(_CB // _C):
                def widx(j2, carry2, jr=jr):
                    sv = src_v[pl.ds(jr * _C + j2 * _L, _L)]
                    dv = dst_v[pl.ds(jr * _C + j2 * _L, _L)]
                    wi_v[pl.ds(jr * _C + j2 * _L, _L)] = (
                        jnp.minimum(sv * N + dv, wclamp))
                    src_s[jr, pl.ds(j2 * _L, _L)] = sv
                    dst_s[jr, pl.ds(j2 * _L, _L)] = dv
                    return carry2
                lax.fori_loop(0, _C // _L, widx, 0)

            pltpu.async_copy(wflat.at[wi_v], we_v, semG0).wait()
            pltpu.async_copy(t_sh.at[dst_v], t_rows, semG0).wait()
            pltpu.async_copy(e_sh.at[src_v], e_rows, semG0).wait()

            for jr in range(_CB // _C):
                def edge16(j2, carry2, jr=jr):
                    base = jr * _C + j2 * _L
                    wv16 = we_v[pl.ds(base, _L)]
                    for l in range(_L):
                        wv = wv16[l]
                        t = t_rows[base + l, :]
                        e = e_rows[base + l, :]
                        mu_c[jr, j2 * _L + l, :] = t * wv
                        ag_c[jr, j2 * _L + l, :] = (1.0 - t * t) * (e * wv)
                    return carry2
                lax.fori_loop(0, _C // _L, edge16, 0)
                pltpu.sync_copy(mu_c.at[jr], mu_sh.at[src_s.at[jr]],
                                add=True)
                pltpu.sync_copy(ag_c.at[jr], ag_sh.at[dst_s.at[jr]],
                                add=True)
            return carry
        lax.fori_loop(0, K, chunk, 0)
        plsc.subcore_barrier()

        pltpu.sync_copy(mu_sh.at[pl.ds(r0, rows)], tb)
        pltpu.sync_copy(tb, mu_out.at[cid, pl.ds(r0, rows)])
        pltpu.sync_copy(ag_sh.at[pl.ds(r0, rows)], tb)
        pltpu.sync_copy(tb, ag_out.at[cid, pl.ds(r0, rows)])

    return pl.kernel(
        body,
        out_type=[jax.ShapeDtypeStruct((_NC, NR, _L), jnp.float32)] * 2,
        mesh=plsc.VectorSubcoreMesh(core_axis_name="c", subcore_axis_name="s",
                                    num_cores=_NC, num_subcores=_NS),
        compiler_params=pltpu.CompilerParams(use_tc_tiling_on_sc=False),
        scratch_types=[
            pltpu.VMEM_SHARED((NR, _L), jnp.float32),
            pltpu.VMEM_SHARED((NR, _L), jnp.float32),
            pltpu.VMEM_SHARED((NR, _L), jnp.float32),
            pltpu.VMEM_SHARED((NR, _L), jnp.float32),
            pltpu.VMEM((NR // _NS, _L), jnp.float32),
            pltpu.VMEM((_CB,), jnp.int32),
            pltpu.VMEM((_CB,), jnp.int32),
            pltpu.VMEM((_CB,), jnp.int32),
            pltpu.VMEM((_CB,), jnp.float32),
            pltpu.VMEM((_CB // _C, _C), jnp.int32),
            pltpu.VMEM((_CB // _C, _C), jnp.int32),
            pltpu.VMEM((_CB, _L), jnp.float32),
            pltpu.VMEM((_CB, _L), jnp.float32),
            pltpu.VMEM((_CB // _C, _C, _L), jnp.float32),
            pltpu.VMEM((_CB // _C, _C, _L), jnp.float32),
            pltpu.SemaphoreType.DMA,
        ],
    )


def _combine_body(mu_p, ag_p, e2, mu_o, gr_o):
    mu_o[...] = mu_p[0] + mu_p[1]
    gr_o[...] = e2[...] - (ag_p[0] + ag_p[1])


def kernel(x, error, w, edge_index):
    N = w.shape[0]
    BN = x.shape[0]
    B = BN // N
    E = edge_index.shape[1]
    assert B == _L
    NR = -(-(N + 1) // (_NS * 8)) * (_NS * 8)
    EW = -(-E // _NW)
    EW = -(-EW // _CB) * _CB
    E_pad = EW * _NW
    pad = E_pad - E

    xT = jnp.pad(x.reshape(B, N).T, ((0, NR - N), (0, 0)))
    eT = jnp.pad(error.reshape(B, N).T, ((0, NR - N), (0, 0)))
    wflat = w.reshape(-1)
    padv = jnp.full((pad,), N, jnp.int32)
    tail = jnp.full((_NW, 2 * _C), N, jnp.int32)
    src = jnp.concatenate(
        [jnp.concatenate([edge_index[0], padv]).reshape(_NW, EW), tail],
        axis=1).reshape(-1)
    dst = jnp.concatenate(
        [jnp.concatenate([edge_index[1], padv]).reshape(_NW, EW), tail],
        axis=1).reshape(-1)

    mu_p, ag_p = _sc_edge_kernel(N, NR, EW)(xT, eT, wflat, src, dst)

    F = NR * _L
    assert F % 128 == 0
    R = F // 128
    muT, gradT = pl.pallas_call(
        _combine_body,
        out_shape=[jax.ShapeDtypeStruct((R, 128), jnp.float32)] * 2,
    )(mu_p.reshape(_NC, R, 128), ag_p.reshape(_NC, R, 128),
      eT.reshape(R, 128))

    mu = muT.reshape(NR, _L)[:N].T.reshape(BN, 1)
    grad = gradT.reshape(NR, _L)[:N].T.reshape(BN, 1)
    return mu, grad
